# baseline (speedup 1.0000x reference)
import functools

import numpy as np
import jax
import jax.numpy as jnp
from jax import lax
from jax.experimental import pallas as pl
from jax.experimental.pallas import tpu as pltpu
from jax.experimental.pallas import tpu_sc as plsc

_TUPLE = 4
_NIN = 4096
_ENT = 16
_BATCH = 1024
_L = 16
_NW = 32
_RPW = _BATCH // _NW
_VPR = _NIN // _L
_CHUNK = 8
_NCHUNK = _RPW // _CHUNK
_UNROLL = 8

_GATHER_DNUMS = lax.GatherDimensionNumbers(
    offset_dims=(), collapsed_slice_dims=(0,), start_index_map=(0,))


def _lane_shuffle(v, idx_const):
    return lax.gather(v, idx_const[:, None],
                      dimension_numbers=_GATHER_DNUMS, slice_sizes=(1,),
                      mode=lax.GatherScatterMode.PROMISE_IN_BOUNDS)


def _body(x_hbm, w_hbm, out_hbm, xbuf0, xbuf1, wbuf, wrepl, resbuf, outbuf,
          sem0, sem1):
    nc = 2
    wid = lax.axis_index("s") * nc + lax.axis_index("c")
    base = wid * _RPW

    lane_iota = lax.iota(jnp.int32, _L)
    pat16 = (jnp.int32(_L) << (lane_iota & 3)).astype(jnp.float32)
    rot1 = (lane_iota + 1) & (_L - 1)
    rot2 = (lane_iota + 2) & (_L - 1)

    pltpu.sync_copy(w_hbm.at[pl.ds(base, _RPW)], wbuf)

    xbufs = [xbuf0, xbuf1]
    sems = [sem0, sem1]
    copies = [None, None]
    copies[0] = pltpu.async_copy(
        x_hbm.at[pl.ds(base, _CHUNK)], xbufs[0], sems[0])

    @plsc.parallel_loop(0, _RPW)
    def repl(r):
        wv = wbuf[r, :]
        for v in range(_ENT):
            wrepl[pl.ds(r * _ENT * _L + v * _L, _L)] = _lane_shuffle(
                wv, jnp.broadcast_to(v, (_L,)))

    for ch in range(_NCHUNK):
        slot = ch % 2
        if ch + 1 < _NCHUNK:
            nslot = (ch + 1) % 2
            copies[nslot] = pltpu.async_copy(
                x_hbm.at[pl.ds(base + (ch + 1) * _CHUNK, _CHUNK)],
                xbufs[nslot], sems[nslot])
        copies[slot].wait()
        xb = xbufs[slot]

        for r in range(_CHUNK):
            wrow_flat = (ch * _CHUNK + r) * (_ENT * _L) + lane_iota
            zero = jnp.zeros((_L,), jnp.float32)

            @plsc.parallel_loop(0, _VPR, unroll=_UNROLL,
                                carry=(zero, zero, zero, zero))
            def accs(i, acc):
                a0, a1, a2, a3 = acc
                v = xb[r, pl.ds(i * _L, _L)]
                t = v * pat16
                s = t + _lane_shuffle(t, rot1)
                s = s + _lane_shuffle(s, rot2)
                g = plsc.load_gather(wrepl, [wrow_flat + s.astype(jnp.int32)])
                return (a1, a2, a3, a0 + g)

            resbuf[ch * _CHUNK + r, :] = (accs[0] + accs[1]) + (accs[2] + accs[3])

    for h in range(_RPW // _L):
        rows = lane_iota + h * _L
        tot = jnp.zeros((_L,), jnp.float32)
        for c in range(0, _L, _TUPLE):
            col = jnp.full((_L,), c, jnp.int32)
            tot = tot + plsc.load_gather(resbuf, [rows, col])
        outbuf[pl.ds(h * _L, _L)] = tot

    pltpu.sync_copy(outbuf, out_hbm.at[pl.ds(base, _RPW)])


@jax.jit
def kernel(x, lut_weights):
    mesh = plsc.VectorSubcoreMesh(core_axis_name="c", subcore_axis_name="s")
    run = pl.kernel(
        _body,
        out_type=jax.ShapeDtypeStruct((_BATCH,), jnp.float32),
        mesh=mesh,
        compiler_params=pltpu.CompilerParams(needs_layout_passes=False),
        scratch_types=[
            pltpu.VMEM((_CHUNK, _NIN), jnp.float32),
            pltpu.VMEM((_CHUNK, _NIN), jnp.float32),
            pltpu.VMEM((_RPW, _ENT), jnp.float32),
            pltpu.VMEM((_RPW * _ENT * _L,), jnp.float32),
            pltpu.VMEM((_RPW, _ENT), jnp.float32),
            pltpu.VMEM((_RPW,), jnp.float32),
            pltpu.SemaphoreType.DMA,
            pltpu.SemaphoreType.DMA,
        ],
    )
    return run(x, lut_weights)


# speedup vs baseline: 1.1981x; 1.1981x over previous
import functools

import numpy as np
import jax
import jax.numpy as jnp
from jax import lax
from jax.experimental import pallas as pl
from jax.experimental.pallas import tpu as pltpu
from jax.experimental.pallas import tpu_sc as plsc

_TUPLE = 4
_NIN = 4096
_ENT = 16
_BATCH = 1024
_L = 16
_NW = 32
_RPW = _BATCH // _NW
_VPR = _NIN // _L
_CHUNK = 8
_NCHUNK = _RPW // _CHUNK
_UNROLL = 16

_GATHER_DNUMS = lax.GatherDimensionNumbers(
    offset_dims=(), collapsed_slice_dims=(0,), start_index_map=(0,))


def _lane_shuffle(v, idx_const):
    return lax.gather(v, idx_const[:, None],
                      dimension_numbers=_GATHER_DNUMS, slice_sizes=(1,),
                      mode=lax.GatherScatterMode.PROMISE_IN_BOUNDS)


def _body(x_hbm, w_hbm, out_hbm, xbuf0, xbuf1, wbuf, histf, resbuf,
          outbuf, sem0, sem1):
    nc = 2
    wid = lax.axis_index("s") * nc + lax.axis_index("c")
    base = wid * _RPW

    lane_iota = lax.iota(jnp.int32, _L)
    pat16 = (jnp.int32(_L) << (lane_iota & 3)).astype(jnp.float32)
    rot1 = (lane_iota + 1) & (_L - 1)
    rot2 = (lane_iota + 2) & (_L - 1)

    pltpu.sync_copy(w_hbm.at[pl.ds(base, _RPW)], wbuf)

    xbufs = [xbuf0, xbuf1]
    sems = [sem0, sem1]
    copies = [None, None]
    copies[0] = pltpu.async_copy(
        x_hbm.at[pl.ds(base, _CHUNK)], xbufs[0], sems[0])

    for ch in range(_NCHUNK):
        slot = ch % 2
        if ch + 1 < _NCHUNK:
            nslot = (ch + 1) % 2
            copies[nslot] = pltpu.async_copy(
                x_hbm.at[pl.ds(base + (ch + 1) * _CHUNK, _CHUNK)],
                xbufs[nslot], sems[nslot])
        copies[slot].wait()
        xb = xbufs[slot]

        @plsc.parallel_loop(0, _CHUNK)
        def row_step(r):
            zero = jnp.zeros((_L,), jnp.float32)
            ones = zero + 1.0
            hbase = r * (_ENT * _L) + lane_iota

            for v in range(_ENT):
                histf[pl.ds(r * (_ENT * _L) + v * _L, _L)] = zero

            @plsc.parallel_loop(0, _VPR, unroll=_UNROLL)
            def scat(i):
                v = xb[r, pl.ds(i * _L, _L)]
                t = v * pat16
                s = t + _lane_shuffle(t, rot1)
                s = s + _lane_shuffle(s, rot2)
                plsc.addupdate_scatter(histf, [hbase + s.astype(jnp.int32)],
                                       ones)

            wv = wbuf[ch * _CHUNK + r, :]
            acc = zero
            for v in range(_ENT):
                cnt = histf[pl.ds(r * (_ENT * _L) + v * _L, _L)]
                acc = acc + cnt * _lane_shuffle(wv, jnp.broadcast_to(v, (_L,)))
            resbuf[ch * _CHUNK + r, :] = acc

    for h in range(_RPW // _L):
        rows = lane_iota + h * _L
        tot = jnp.zeros((_L,), jnp.float32)
        for c in range(0, _L, _TUPLE):
            col = jnp.full((_L,), c, jnp.int32)
            tot = tot + plsc.load_gather(resbuf, [rows, col])
        outbuf[pl.ds(h * _L, _L)] = tot

    pltpu.sync_copy(outbuf, out_hbm.at[pl.ds(base, _RPW)])


@jax.jit
def kernel(x, lut_weights):
    mesh = plsc.VectorSubcoreMesh(core_axis_name="c", subcore_axis_name="s")
    run = pl.kernel(
        _body,
        out_type=jax.ShapeDtypeStruct((_BATCH,), jnp.float32),
        mesh=mesh,
        compiler_params=pltpu.CompilerParams(needs_layout_passes=False),
        scratch_types=[
            pltpu.VMEM((_CHUNK, _NIN), jnp.float32),
            pltpu.VMEM((_CHUNK, _NIN), jnp.float32),
            pltpu.VMEM((_RPW, _ENT), jnp.float32),
            pltpu.VMEM((_CHUNK * _ENT * _L,), jnp.float32),
            pltpu.VMEM((_RPW, _ENT), jnp.float32),
            pltpu.VMEM((_RPW,), jnp.float32),
            pltpu.SemaphoreType.DMA,
            pltpu.SemaphoreType.DMA,
        ],
    )
    return run(x, lut_weights)
